# baseline (speedup 1.0000x reference)
import functools

import jax
import jax.numpy as jnp
from jax import lax
from jax.experimental import pallas as pl
from jax.experimental.pallas import tpu as pltpu
from jax.experimental.pallas import tpu_sc as plsc

B, N, D = 4, 8192, 2048
TC_B = 2
SC_B = B - TC_B

NC, NS, L = 2, 16, 16
NW = NC * NS
COLS_PER_W = (SC_B * D) // NW
CW = COLS_PER_W
G = CW // L
RB = 256
NRC = N // RB
UNROLL = 4

_mesh = plsc.VectorSubcoreMesh(core_axis_name="c", subcore_axis_name="s")


@functools.partial(
    pl.kernel,
    out_type=jax.ShapeDtypeStruct((SC_B * D,), jnp.int32),
    mesh=_mesh,
    scratch_types=[
        pltpu.VMEM((RB, CW), jnp.float32),
        pltpu.VMEM((RB, CW), jnp.float32),
        pltpu.VMEM((COLS_PER_W,), jnp.int32),
        pltpu.SemaphoreType.DMA,
        pltpu.SemaphoreType.DMA,
    ],
)
def _argmin_sc(x_hbm, out_hbm, buf0, buf1, outv, sem0, sem1):
    wid = lax.axis_index("c") * NS + lax.axis_index("s")
    base = wid * COLS_PER_W
    b = TC_B + base // D
    j0 = base % D

    bufs = (buf0, buf1)
    sems = (sem0, sem1)

    def copy(rc, ph):
        return pltpu.make_async_copy(
            x_hbm.at[b, pl.ds(rc * RB, RB), pl.ds(j0, CW)],
            bufs[ph], sems[ph])

    def compute(buf, r0, carry):
        def row_body(r, carry2):
            mv, mi = carry2
            rv = jnp.full((L,), r0 + r, jnp.int32)
            mv2, mi2 = [], []
            for g in range(G):
                v = buf[r, g * L:(g + 1) * L]
                p = v < mv[g]
                mv2.append(jnp.minimum(v, mv[g]))
                mi2.append(jnp.where(p, rv, mi[g]))
            return (tuple(mv2), tuple(mi2))

        return lax.fori_loop(0, RB, row_body, carry, unroll=UNROLL)

    copy(0, 0).start()

    def pair_body(i, carry):
        rc0 = 2 * i
        copy(rc0 + 1, 1).start()
        copy(rc0, 0).wait()
        carry = compute(buf0, rc0 * RB, carry)

        @pl.when(rc0 + 2 < NRC)
        def _():
            copy(rc0 + 2, 0).start()

        copy(rc0 + 1, 1).wait()
        carry = compute(buf1, (rc0 + 1) * RB, carry)
        return carry

    init = (
        tuple(jnp.full((L,), jnp.inf, jnp.float32) for _ in range(G)),
        tuple(jnp.zeros((L,), jnp.int32) for _ in range(G)),
    )
    _, minis = lax.fori_loop(0, NRC // 2, pair_body, init)
    for g in range(G):
        outv[g * L:(g + 1) * L] = minis[g]

    pltpu.sync_copy(outv, out_hbm.at[pl.ds(base, COLS_PER_W)])


RC = 512
NRC_TC = N // RC
RCH = 16


def _argmin_tc_body(x_ref, o_ref, mv_ref, mi_ref, iota_ref):
    b = pl.program_id(0)
    i = pl.program_id(1)

    @pl.when(jnp.logical_and(b == 0, i == 0))
    def _():
        iota_ref[...] = lax.broadcasted_iota(
            jnp.int32, (RC, D), 0).astype(jnp.float32)

    @pl.when(i == 0)
    def _():
        mv_ref[...] = jnp.full((1, D), jnp.inf, jnp.float32)
        mi_ref[...] = jnp.zeros((1, D), jnp.float32)

    xb = x_ref[0]
    cm = jnp.min(xb, axis=0, keepdims=True)
    big = jnp.float32(2 * N)
    ci = jnp.full((1, D), big, jnp.float32)
    for c in range(RC // RCH):
        xc = xb[c * RCH:(c + 1) * RCH]
        ic = iota_ref[c * RCH:(c + 1) * RCH]
        cic = jnp.min(jnp.where(xc == cm, ic, big), axis=0, keepdims=True)
        ci = jnp.minimum(ci, cic)
    ci = ci + jnp.float32(1.0) * (i * RC)
    mv = mv_ref[...]
    p = cm < mv
    mv_ref[...] = jnp.minimum(cm, mv)
    mi_ref[...] = jnp.where(p, ci, mi_ref[...])

    @pl.when(i == NRC_TC - 1)
    def _():
        o_ref[...] = mi_ref[...].astype(jnp.int32).reshape(1, 1, D)


_argmin_tc = pl.pallas_call(
    _argmin_tc_body,
    grid=(TC_B, NRC_TC),
    in_specs=[pl.BlockSpec((1, RC, D), lambda b, i: (b, i, 0))],
    out_specs=pl.BlockSpec((1, 1, D), lambda b, i: (b, 0, 0)),
    out_shape=jax.ShapeDtypeStruct((TC_B, 1, D), jnp.int32),
    scratch_shapes=[
        pltpu.VMEM((1, D), jnp.float32),
        pltpu.VMEM((1, D), jnp.float32),
        pltpu.VMEM((RC, D), jnp.float32),
    ],
)


def kernel(x):
    sc_out = _argmin_sc(x).reshape(SC_B, D)
    tc_out = _argmin_tc(x).reshape(TC_B, D)
    out = jnp.concatenate([tc_out, sc_out], axis=0)
    return out.astype(jnp.int64)
